# baseline (speedup 1.0000x reference)
import functools

import jax
import jax.numpy as jnp
from jax import lax
from jax.experimental import pallas as pl
from jax.experimental.pallas import tpu as pltpu
from jax.experimental.pallas import tpu_sc as plsc

N = 10000
NP = 10240
E = 320000
EL = 320000
EP = 327680
D_IN = 128
D_HID = 128
D_OUT = 64

NC, NS = 2, 16
NW = NC * NS
RPT = NP // NS

C = 128
NCH = EP // NW // C
ELW = EP // NW

BM = 1024


def _sc_mesh():
  return plsc.VectorSubcoreMesh(core_axis_name="c", subcore_axis_name="s")


@functools.partial(
    pl.kernel,
    mesh=_sc_mesh(),
    out_type=jax.ShapeDtypeStruct((NC * NP,), jnp.float32),
    scratch_types=[
        pltpu.VMEM((NCH, C), jnp.int32),
        pltpu.VMEM((C,), jnp.float32),
        pltpu.VMEM_SHARED((NP,), jnp.float32),
    ],
)
def _deg_kernel(dst_hbm, ones_hbm, zeros_hbm, out_hbm, didx_v, ones_v, acc_s):
  cid = lax.axis_index("c")
  sid = lax.axis_index("s")
  wid = sid * NC + cid
  pltpu.sync_copy(zeros_hbm, acc_s.at[pl.ds(sid * RPT, RPT)])
  pltpu.sync_copy(ones_hbm, ones_v)
  pltpu.sync_copy(dst_hbm.at[pl.ds(wid * NCH, NCH)], didx_v)
  plsc.subcore_barrier()

  def body(j, carry):
    pltpu.sync_copy(ones_v, acc_s.at[didx_v.at[j]], add=True)
    return carry

  lax.fori_loop(0, NCH, body, 0)
  plsc.subcore_barrier()
  pltpu.sync_copy(acc_s.at[pl.ds(sid * RPT, RPT)],
                  out_hbm.at[pl.ds(cid * NP + sid * RPT, RPT)])


NCHH = 40
NCHT = EP // NS // C


@functools.partial(
    pl.kernel,
    mesh=_sc_mesh(),
    out_type=jax.ShapeDtypeStruct((NP, D_HID), jnp.float32),
    scratch_types=[
        pltpu.VMEM((NCHH, C), jnp.int32),
        pltpu.VMEM((NCHH, C), jnp.int32),
        pltpu.VMEM((C, D_HID), jnp.float32),
        pltpu.VMEM((C, D_HID), jnp.float32),
        pltpu.VMEM_SHARED((NP, D_HID), jnp.float32),
        pltpu.SemaphoreType.DMA,
        pltpu.SemaphoreType.DMA,
    ],
)
def _agg_kernel(p_hbm, src_hbm, dst_hbm, zeros_hbm, out_hbm,
                sidx_v, didx_v, rows0_v, rows1_v, acc_s, gsem0, gsem1):
  cid = lax.axis_index("c")
  sid = lax.axis_index("s")

  def body(t, carry):
    j0 = 2 * t
    j1 = j0 + 1
    pltpu.async_copy(p_hbm.at[sidx_v.at[j1]], rows1_v, gsem1)
    pltpu.make_async_copy(p_hbm.at[sidx_v.at[j0]], rows0_v, gsem0).wait()
    pltpu.sync_copy(rows0_v, acc_s.at[didx_v.at[j0]], add=True)

    @pl.when(t < NCHH // 2 - 1)
    def _():
      pltpu.async_copy(p_hbm.at[sidx_v.at[j0 + 2]], rows0_v, gsem0)

    pltpu.make_async_copy(p_hbm.at[sidx_v.at[j1]], rows1_v, gsem1).wait()
    pltpu.sync_copy(rows1_v, acc_s.at[didx_v.at[j1]], add=True)
    return carry

  @pl.when(cid == 0)
  def _():
    pltpu.sync_copy(zeros_hbm, acc_s.at[pl.ds(sid * RPT, RPT)])
    plsc.subcore_barrier()
    for h in range(NCHT // NCHH):
      base = sid * NCHT + h * NCHH
      pltpu.sync_copy(src_hbm.at[pl.ds(base, NCHH)], sidx_v)
      pltpu.sync_copy(dst_hbm.at[pl.ds(base, NCHH)], didx_v)
      pltpu.async_copy(p_hbm.at[sidx_v.at[0]], rows0_v, gsem0)
      lax.fori_loop(0, NCHH // 2, body, 0)
    plsc.subcore_barrier()
    pltpu.sync_copy(acc_s.at[pl.ds(sid * RPT, RPT)],
                    out_hbm.at[pl.ds(sid * RPT, RPT)])


@functools.partial(
    pl.kernel,
    mesh=_sc_mesh(),
    out_type=jax.ShapeDtypeStruct((EP,), jnp.float32),
    compiler_params=pltpu.CompilerParams(needs_layout_passes=False),
    scratch_types=[
        pltpu.VMEM((NP,), jnp.float32),
        pltpu.VMEM((NP,), jnp.float32),
        pltpu.VMEM((NP,), jnp.float32),
        pltpu.VMEM((ELW,), jnp.int32),
        pltpu.VMEM((ELW,), jnp.int32),
        pltpu.VMEM((ELW,), jnp.int32),
        pltpu.VMEM((ELW,), jnp.float32),
    ],
)
def _decode_kernel(s_hbm, a_hbm, b_hbm, c_hbm, out_hbm,
                   s0_v, s1_v, s2_v, a_v, b_v, c_v, o_v):
  cid = lax.axis_index("c")
  sid = lax.axis_index("s")
  wid = sid * NC + cid
  base = wid * ELW
  pltpu.sync_copy(s_hbm.at[pl.ds(0, NP)], s0_v)
  pltpu.sync_copy(s_hbm.at[pl.ds(NP, NP)], s1_v)
  pltpu.sync_copy(s_hbm.at[pl.ds(2 * NP, NP)], s2_v)
  pltpu.sync_copy(a_hbm.at[pl.ds(base, ELW)], a_v)
  pltpu.sync_copy(b_hbm.at[pl.ds(base, ELW)], b_v)
  pltpu.sync_copy(c_hbm.at[pl.ds(base, ELW)], c_v)

  def body(i, carry):
    off = i * 16
    va = plsc.load_gather(s0_v, [a_v[pl.ds(off, 16)]])
    vb = plsc.load_gather(s1_v, [b_v[pl.ds(off, 16)]])
    vc = plsc.load_gather(s2_v, [c_v[pl.ds(off, 16)]])
    o_v[pl.ds(off, 16)] = va + vb + vc
    return carry

  lax.fori_loop(0, ELW // 16, body, 0)
  pltpu.sync_copy(o_v, out_hbm.at[pl.ds(base, ELW)])


def _k1_body(degp_ref, x_ref, w1_ref, o_ref):
  deg = degp_ref[0, :] + degp_ref[1, :] + 1.0
  dis = lax.rsqrt(deg)
  h = jnp.dot(x_ref[...], w1_ref[...], preferred_element_type=jnp.float32)
  o_ref[...] = h * dis[:, None]


def _k2_body(degp_ref, aggp_ref, p1_ref, b1_ref, o_ref):
  deg = degp_ref[0, :] + degp_ref[1, :] + 1.0
  dis = lax.rsqrt(deg)[:, None]
  t = (aggp_ref[...] + p1_ref[...]) * dis + b1_ref[...]
  o_ref[...] = jnp.maximum(t, 0.0) * dis


def _k3_body(degp_ref, aggp_ref, q1_ref, b2_ref, w2_ref, wt3_ref, bias3_ref,
             o_ref):
  deg = degp_ref[0, :] + degp_ref[1, :] + 1.0
  dis = lax.rsqrt(deg)[:, None]
  u = aggp_ref[...] + q1_ref[...]
  h = jnp.dot(u, w2_ref[...], preferred_element_type=jnp.float32)
  z2 = h * dis + b2_ref[...]
  s = lax.dot_general(wt3_ref[...], z2, (((1,), (1,)), ((), ())),
                      preferred_element_type=jnp.float32)
  o_ref[...] = s + bias3_ref[...]


def _tc_k1(degp, x_pad, W1):
  return pl.pallas_call(
      _k1_body,
      grid=(NP // BM,),
      in_specs=[
          pl.BlockSpec((NC, BM), lambda i: (0, i)),
          pl.BlockSpec((BM, D_IN), lambda i: (i, 0)),
          pl.BlockSpec((D_IN, D_HID), lambda i: (0, 0)),
      ],
      out_specs=pl.BlockSpec((BM, D_HID), lambda i: (i, 0)),
      out_shape=jax.ShapeDtypeStruct((NP, D_HID), jnp.float32),
  )(degp, x_pad, W1)


def _tc_k2(degp, agg1, p1, b1r):
  return pl.pallas_call(
      _k2_body,
      grid=(NP // BM,),
      in_specs=[
          pl.BlockSpec((NC, BM), lambda i: (0, i)),
          pl.BlockSpec((BM, D_HID), lambda i: (i, 0)),
          pl.BlockSpec((BM, D_HID), lambda i: (i, 0)),
          pl.BlockSpec((1, D_HID), lambda i: (0, 0)),
      ],
      out_specs=pl.BlockSpec((BM, D_HID), lambda i: (i, 0)),
      out_shape=jax.ShapeDtypeStruct((NP, D_HID), jnp.float32),
  )(degp, agg1, p1, b1r)


def _tc_k3(degp, agg2, q1, b2r, W2, Wt3, bias3):
  return pl.pallas_call(
      _k3_body,
      grid=(NP // BM,),
      in_specs=[
          pl.BlockSpec((NC, BM), lambda i: (0, i)),
          pl.BlockSpec((BM, D_HID), lambda i: (i, 0)),
          pl.BlockSpec((BM, D_HID), lambda i: (i, 0)),
          pl.BlockSpec((1, D_OUT), lambda i: (0, 0)),
          pl.BlockSpec((D_HID, D_OUT), lambda i: (0, 0)),
          pl.BlockSpec((3, D_OUT), lambda i: (0, 0)),
          pl.BlockSpec((3, 1), lambda i: (0, 0)),
      ],
      out_specs=pl.BlockSpec((3, BM), lambda i: (0, i)),
      out_shape=jax.ShapeDtypeStruct((3, NP), jnp.float32),
  )(degp, agg2, q1, b2r, W2, Wt3, bias3)


def kernel(x, edge_index, edge_label_index, W1, b1, W2, b2, Wt, bt):
  x_pad = jnp.pad(x, ((0, NP - N), (0, 0)))
  pad_e = jnp.full((EP - E,), NP - 1, jnp.int32)
  src_f = jnp.concatenate([edge_index[0], pad_e]).reshape(NW * NCH, C)
  dst_f = jnp.concatenate([edge_index[1], pad_e]).reshape(NW * NCH, C)
  pad_l = jnp.zeros((EP - EL,), jnp.int32)
  a_f = jnp.concatenate([edge_label_index[0], pad_l])
  b_f = jnp.concatenate([edge_label_index[1], pad_l])
  c_f = jnp.concatenate([edge_label_index[2], pad_l])
  ones_c = jnp.ones((C,), jnp.float32)
  zeros_1 = jnp.zeros((RPT,), jnp.float32)
  zeros_h = jnp.zeros((RPT, D_HID), jnp.float32)
  b1r = b1.reshape(1, D_HID)
  b2r = b2.reshape(1, D_OUT)
  Wt3 = Wt[:, 0].reshape(3, D_OUT)
  bias3 = jnp.concatenate([bt, jnp.zeros((2,), jnp.float32)]).reshape(3, 1)

  degp = _deg_kernel(dst_f, ones_c, zeros_1).reshape(NC, NP)
  p1 = _tc_k1(degp, x_pad, W1)
  agg1 = _agg_kernel(p1, src_f, dst_f, zeros_h)
  q1 = _tc_k2(degp, agg1, p1, b1r)
  agg2 = _agg_kernel(q1, src_f, dst_f, zeros_h)
  s = _tc_k3(degp, agg2, q1, b2r, W2, Wt3, bias3)
  logits = _decode_kernel(s.reshape(3 * NP), a_f, b_f, c_f)
  return logits[:EL]


# speedup vs baseline: 1.0659x; 1.0659x over previous
import functools

import jax
import jax.numpy as jnp
from jax import lax
from jax.experimental import pallas as pl
from jax.experimental.pallas import tpu as pltpu
from jax.experimental.pallas import tpu_sc as plsc

N = 10000
NP = 10240
E = 320000
EL = 320000
EP = 327680
D_IN = 128
D_HID = 128
D_OUT = 64

NC, NS = 2, 16
NW = NC * NS
RPT = NP // NS

C = 128
NCH = EP // NW // C
ELW = EP // NW

BM = 1024


def _sc_mesh():
  return plsc.VectorSubcoreMesh(core_axis_name="c", subcore_axis_name="s")


@functools.partial(
    pl.kernel,
    mesh=_sc_mesh(),
    out_type=jax.ShapeDtypeStruct((NC * NP,), jnp.float32),
    scratch_types=[
        pltpu.VMEM((NCH, C), jnp.int32),
        pltpu.VMEM((C,), jnp.float32),
        pltpu.VMEM_SHARED((NP,), jnp.float32),
    ],
)
def _deg_kernel(dst_hbm, ones_hbm, zeros_hbm, out_hbm, didx_v, ones_v, acc_s):
  cid = lax.axis_index("c")
  sid = lax.axis_index("s")
  wid = sid * NC + cid
  pltpu.sync_copy(zeros_hbm, acc_s.at[pl.ds(sid * RPT, RPT)])
  pltpu.sync_copy(ones_hbm, ones_v)
  pltpu.sync_copy(dst_hbm.at[pl.ds(wid * NCH, NCH)], didx_v)
  plsc.subcore_barrier()

  def body(j, carry):
    pltpu.sync_copy(ones_v, acc_s.at[didx_v.at[j]], add=True)
    return carry

  lax.fori_loop(0, NCH, body, 0)
  plsc.subcore_barrier()
  pltpu.sync_copy(acc_s.at[pl.ds(sid * RPT, RPT)],
                  out_hbm.at[pl.ds(cid * NP + sid * RPT, RPT)])


DH = D_HID // 2
NCHH = 40
NCH0 = 120
NCH1 = 40
TOT_CH = EP // C


@functools.partial(
    pl.kernel,
    mesh=_sc_mesh(),
    out_type=jax.ShapeDtypeStruct((NC, NP, D_HID), jnp.float32),
    scratch_types=[
        pltpu.VMEM((NCHH, C), jnp.int32),
        pltpu.VMEM((NCHH, C), jnp.int32),
        pltpu.VMEM((C, D_HID), jnp.float32),
        pltpu.VMEM((C, D_HID), jnp.float32),
        pltpu.VMEM_SHARED((NP, D_HID), jnp.float32),
        pltpu.SemaphoreType.DMA,
        pltpu.SemaphoreType.DMA,
    ],
)
def _agg_kernel(p_hbm, src_hbm, dst_hbm, zeros_hbm, out_hbm,
                sidx_v, didx_v, rows0_v, rows1_v, acc_s, gsem0, gsem1):
  cid = lax.axis_index("c")
  sid = lax.axis_index("s")
  pltpu.sync_copy(zeros_hbm, acc_s.at[pl.ds(sid * RPT, RPT)])
  plsc.subcore_barrier()

  def body(t, carry):
    j0 = 2 * t
    j1 = j0 + 1
    pltpu.async_copy(p_hbm.at[sidx_v.at[j1]], rows1_v, gsem1)
    pltpu.make_async_copy(p_hbm.at[sidx_v.at[j0]], rows0_v, gsem0).wait()
    pltpu.sync_copy(rows0_v, acc_s.at[didx_v.at[j0]], add=True)

    @pl.when(t < NCHH // 2 - 1)
    def _():
      pltpu.async_copy(p_hbm.at[sidx_v.at[j0 + 2]], rows0_v, gsem0)

    pltpu.make_async_copy(p_hbm.at[sidx_v.at[j1]], rows1_v, gsem1).wait()
    pltpu.sync_copy(rows1_v, acc_s.at[didx_v.at[j1]], add=True)
    return carry

  def edge_loop(first_chunk, npass):
    for h in range(npass):
      base = first_chunk + h * NCHH
      pltpu.sync_copy(src_hbm.at[pl.ds(base, NCHH)], sidx_v)
      pltpu.sync_copy(dst_hbm.at[pl.ds(base, NCHH)], didx_v)
      pltpu.async_copy(p_hbm.at[sidx_v.at[0]], rows0_v, gsem0)
      lax.fori_loop(0, NCHH // 2, body, 0)

  @pl.when(cid == 0)
  def _():
    edge_loop(sid * NCH0, NCH0 // NCHH)

  @pl.when(cid == 1)
  def _():
    edge_loop(16 * NCH0 + sid * NCH1, NCH1 // NCHH)

  plsc.subcore_barrier()
  pltpu.sync_copy(acc_s.at[pl.ds(sid * RPT, RPT)],
                  out_hbm.at[cid, pl.ds(sid * RPT, RPT)])


@functools.partial(
    pl.kernel,
    mesh=_sc_mesh(),
    out_type=jax.ShapeDtypeStruct((EP,), jnp.float32),
    compiler_params=pltpu.CompilerParams(needs_layout_passes=False),
    scratch_types=[
        pltpu.VMEM((NP,), jnp.float32),
        pltpu.VMEM((NP,), jnp.float32),
        pltpu.VMEM((NP,), jnp.float32),
        pltpu.VMEM((ELW,), jnp.int32),
        pltpu.VMEM((ELW,), jnp.int32),
        pltpu.VMEM((ELW,), jnp.int32),
        pltpu.VMEM((ELW,), jnp.float32),
    ],
)
def _decode_kernel(s_hbm, a_hbm, b_hbm, c_hbm, out_hbm,
                   s0_v, s1_v, s2_v, a_v, b_v, c_v, o_v):
  cid = lax.axis_index("c")
  sid = lax.axis_index("s")
  wid = sid * NC + cid
  base = wid * ELW
  pltpu.sync_copy(s_hbm.at[pl.ds(0, NP)], s0_v)
  pltpu.sync_copy(s_hbm.at[pl.ds(NP, NP)], s1_v)
  pltpu.sync_copy(s_hbm.at[pl.ds(2 * NP, NP)], s2_v)
  pltpu.sync_copy(a_hbm.at[pl.ds(base, ELW)], a_v)
  pltpu.sync_copy(b_hbm.at[pl.ds(base, ELW)], b_v)
  pltpu.sync_copy(c_hbm.at[pl.ds(base, ELW)], c_v)

  def body(i, carry):
    off = i * 16
    va = plsc.load_gather(s0_v, [a_v[pl.ds(off, 16)]])
    vb = plsc.load_gather(s1_v, [b_v[pl.ds(off, 16)]])
    vc = plsc.load_gather(s2_v, [c_v[pl.ds(off, 16)]])
    o_v[pl.ds(off, 16)] = va + vb + vc
    return carry

  lax.fori_loop(0, ELW // 16, body, 0)
  pltpu.sync_copy(o_v, out_hbm.at[pl.ds(base, ELW)])


def _k1_body(degp_ref, x_ref, w1_ref, o_ref):
  deg = degp_ref[0, :] + degp_ref[1, :] + 1.0
  dis = lax.rsqrt(deg)
  h = jnp.dot(x_ref[...], w1_ref[...], preferred_element_type=jnp.float32)
  o_ref[...] = h * dis[:, None]


def _k2_body(degp_ref, aggp_ref, p1_ref, b1_ref, o_ref):
  deg = degp_ref[0, :] + degp_ref[1, :] + 1.0
  dis = lax.rsqrt(deg)[:, None]
  t = (aggp_ref[0] + aggp_ref[1] + p1_ref[...]) * dis + b1_ref[...]
  o_ref[...] = jnp.maximum(t, 0.0) * dis


def _k3_body(degp_ref, aggp_ref, q1_ref, b2_ref, w2_ref, wt3_ref, bias3_ref,
             o_ref):
  deg = degp_ref[0, :] + degp_ref[1, :] + 1.0
  dis = lax.rsqrt(deg)[:, None]
  u = aggp_ref[0] + aggp_ref[1] + q1_ref[...]
  h = jnp.dot(u, w2_ref[...], preferred_element_type=jnp.float32)
  z2 = h * dis + b2_ref[...]
  s = lax.dot_general(wt3_ref[...], z2, (((1,), (1,)), ((), ())),
                      preferred_element_type=jnp.float32)
  o_ref[...] = s + bias3_ref[...]


def _tc_k1(degp, x_pad, W1):
  return pl.pallas_call(
      _k1_body,
      grid=(NP // BM,),
      in_specs=[
          pl.BlockSpec((NC, BM), lambda i: (0, i)),
          pl.BlockSpec((BM, D_IN), lambda i: (i, 0)),
          pl.BlockSpec((D_IN, D_HID), lambda i: (0, 0)),
      ],
      out_specs=pl.BlockSpec((BM, D_HID), lambda i: (i, 0)),
      out_shape=jax.ShapeDtypeStruct((NP, D_HID), jnp.float32),
  )(degp, x_pad, W1)


def _tc_k2(degp, agg1, p1, b1r):
  return pl.pallas_call(
      _k2_body,
      grid=(NP // BM,),
      in_specs=[
          pl.BlockSpec((NC, BM), lambda i: (0, i)),
          pl.BlockSpec((NC, BM, D_HID), lambda i: (0, i, 0)),
          pl.BlockSpec((BM, D_HID), lambda i: (i, 0)),
          pl.BlockSpec((1, D_HID), lambda i: (0, 0)),
      ],
      out_specs=pl.BlockSpec((BM, D_HID), lambda i: (i, 0)),
      out_shape=jax.ShapeDtypeStruct((NP, D_HID), jnp.float32),
  )(degp, agg1, p1, b1r)


def _tc_k3(degp, agg2, q1, b2r, W2, Wt3, bias3):
  return pl.pallas_call(
      _k3_body,
      grid=(NP // BM,),
      in_specs=[
          pl.BlockSpec((NC, BM), lambda i: (0, i)),
          pl.BlockSpec((NC, BM, D_HID), lambda i: (0, i, 0)),
          pl.BlockSpec((BM, D_HID), lambda i: (i, 0)),
          pl.BlockSpec((1, D_OUT), lambda i: (0, 0)),
          pl.BlockSpec((D_HID, D_OUT), lambda i: (0, 0)),
          pl.BlockSpec((3, D_OUT), lambda i: (0, 0)),
          pl.BlockSpec((3, 1), lambda i: (0, 0)),
      ],
      out_specs=pl.BlockSpec((3, BM), lambda i: (0, i)),
      out_shape=jax.ShapeDtypeStruct((3, NP), jnp.float32),
  )(degp, agg2, q1, b2r, W2, Wt3, bias3)


def kernel(x, edge_index, edge_label_index, W1, b1, W2, b2, Wt, bt):
  x_pad = jnp.pad(x, ((0, NP - N), (0, 0)))
  pad_e = jnp.full((EP - E,), NP - 1, jnp.int32)
  src_f = jnp.concatenate([edge_index[0], pad_e]).reshape(NW * NCH, C)
  dst_f = jnp.concatenate([edge_index[1], pad_e]).reshape(NW * NCH, C)
  pad_l = jnp.zeros((EP - EL,), jnp.int32)
  a_f = jnp.concatenate([edge_label_index[0], pad_l])
  b_f = jnp.concatenate([edge_label_index[1], pad_l])
  c_f = jnp.concatenate([edge_label_index[2], pad_l])
  ones_c = jnp.ones((C,), jnp.float32)
  zeros_1 = jnp.zeros((RPT,), jnp.float32)
  zeros_h = jnp.zeros((RPT, D_HID), jnp.float32)
  b1r = b1.reshape(1, D_HID)
  b2r = b2.reshape(1, D_OUT)
  Wt3 = Wt[:, 0].reshape(3, D_OUT)
  bias3 = jnp.concatenate([bt, jnp.zeros((2,), jnp.float32)]).reshape(3, 1)

  degp = _deg_kernel(dst_f, ones_c, zeros_1).reshape(NC, NP)
  p1 = _tc_k1(degp, x_pad, W1)
  agg1 = _agg_kernel(p1, src_f, dst_f, zeros_h)
  q1 = _tc_k2(degp, agg1, p1, b1r)
  agg2 = _agg_kernel(q1, src_f, dst_f, zeros_h)
  s = _tc_k3(degp, agg2, q1, b2r, W2, Wt3, bias3)
  logits = _decode_kernel(s.reshape(3 * NP), a_f, b_f, c_f)
  return logits[:EL]
